# Initial kernel scaffold; baseline (speedup 1.0000x reference)
import functools

import jax
import jax.numpy as jnp
from jax import lax
from jax.experimental import pallas as pl
from jax.experimental.pallas import tpu as pltpu
from jax.experimental.pallas import tpu_sc as plsc

_N = 10000
_E = 320000
_D = 128
_NPAD = 10240
_K = 80
_NSUB = 16
_EPS = _E // _NSUB
_NWIN = _EPS // _K
_ZROWS = _NPAD // _NSUB
_OROWS = _N // _NSUB
_RB = 2000


def _segsum_body(tab_hbm, src_hbm, dst_hbm, out_hbm, srcv, dstv, rows, acc,
                 sem):
  c = lax.axis_index("c")
  s = lax.axis_index("s")

  zero = jnp.zeros((16,), jnp.float32)

  @pl.loop(0, _K)
  def _(r):
    @pl.loop(0, _D, step=16)
    def _(col):
      rows[r, pl.ds(col, 16)] = zero

  @pl.loop(0, _ZROWS, step=_K)
  def _(r0):
    pltpu.sync_copy(rows, acc.at[pl.ds(s * _ZROWS + r0, _K)])

  plsc.subcore_barrier()

  row_off = c * _N

  @pl.loop(0, _NWIN)
  def _(w):
    base = s * _EPS + w * _K
    pltpu.sync_copy(src_hbm.at[pl.ds(base, _K)], srcv.at[0])
    pltpu.sync_copy(dst_hbm.at[pl.ds(base, _K)], dstv.at[0])

    @pl.loop(0, _K, step=16)
    def _(j):
      srcv[0, pl.ds(j, 16)] = srcv[0, pl.ds(j, 16)] + row_off

    pltpu.async_copy(tab_hbm.at[srcv.at[0]], rows, sem).wait()
    pltpu.sync_copy(rows, acc.at[dstv.at[0]], add=True)

  plsc.subcore_barrier()
  pltpu.sync_copy(acc.at[pl.ds(s * _OROWS, _OROWS)],
                  out_hbm.at[pl.ds(c * _N + s * _OROWS, _OROWS)])


def _segsum2(tables, src, dst):
  kern = pl.kernel(
      _segsum_body,
      out_type=jax.ShapeDtypeStruct((2 * _N, _D), jnp.float32),
      mesh=plsc.VectorSubcoreMesh(core_axis_name="c", subcore_axis_name="s"),
      scratch_types=[
          pltpu.VMEM((1, _K), jnp.int32),
          pltpu.VMEM((1, _K), jnp.int32),
          pltpu.VMEM((_K, _D), jnp.float32),
          pltpu.VMEM_SHARED((_NPAD, _D), jnp.float32),
          pltpu.SemaphoreType.DMA,
      ],
  )
  return kern(tables, src, dst)


def _l1_body(x_ref, agg_ref, w1_ref, b1_ref, w2_ref, b2_ref, out_ref):
  m = x_ref[0] + agg_ref[0]
  u = jnp.maximum(
      jnp.dot(m, w1_ref[0], preferred_element_type=jnp.float32) + b1_ref[0],
      0.0)
  h = jnp.maximum(
      jnp.dot(u, w2_ref[0], preferred_element_type=jnp.float32) + b2_ref[0],
      0.0)
  out_ref[0, 0] = h[:, :_D]
  out_ref[0, 1] = h[:, _D:]


def _l2_body(h1_ref, agg_ref, w1_ref, b1_ref, w2_ref, b2_ref, out_ref):
  m_lo = h1_ref[0, 0] + agg_ref[0, 0]
  m_hi = h1_ref[0, 1] + agg_ref[0, 1]
  u = (jnp.dot(m_lo, w1_ref[0, :_D], preferred_element_type=jnp.float32)
       + jnp.dot(m_hi, w1_ref[0, _D:], preferred_element_type=jnp.float32)
       + b1_ref[0])
  u = jnp.maximum(u, 0.0)
  out_ref[0] = (jnp.dot(u, w2_ref[0], preferred_element_type=jnp.float32)
                + b2_ref[0])


def _proj_body(h_ref, w1_ref, b1_ref, g_ref, be_ref, w2_ref, b2_ref, out_ref):
  u = jnp.dot(h_ref[0], w1_ref[0], preferred_element_type=jnp.float32) \
      + b1_ref[0]
  mean = jnp.mean(u, axis=0)
  var = jnp.mean((u - mean) ** 2, axis=0)
  un = g_ref[0] * (u - mean) / jnp.sqrt(var + 1e-5) + be_ref[0]
  ur = jnp.maximum(un, 0.0)
  out_ref[0] = (jnp.dot(ur, w2_ref[0], preferred_element_type=jnp.float32)
                + b2_ref[0])


def _l1(x_st, agg1, w1, b1, w2, b2):
  grid = (2, _N // _RB)
  return pl.pallas_call(
      _l1_body,
      grid=grid,
      in_specs=[
          pl.BlockSpec((1, _RB, _D), lambda e, r: (e, r, 0)),
          pl.BlockSpec((1, _RB, _D), lambda e, r: (e, r, 0)),
          pl.BlockSpec((1, _D, 2 * _D), lambda e, r: (e, 0, 0)),
          pl.BlockSpec((1, 2 * _D), lambda e, r: (e, 0)),
          pl.BlockSpec((1, 2 * _D, 2 * _D), lambda e, r: (e, 0, 0)),
          pl.BlockSpec((1, 2 * _D), lambda e, r: (e, 0)),
      ],
      out_specs=pl.BlockSpec((1, 2, _RB, _D), lambda e, r: (e, 0, r, 0)),
      out_shape=jax.ShapeDtypeStruct((2, 2, _N, _D), jnp.float32),
  )(x_st, agg1, w1, b1, w2, b2)


def _l2(h1, agg2, w1, b1, w2, b2):
  grid = (2, _N // _RB)
  return pl.pallas_call(
      _l2_body,
      grid=grid,
      in_specs=[
          pl.BlockSpec((1, 2, _RB, _D), lambda e, r: (e, 0, r, 0)),
          pl.BlockSpec((1, 2, _RB, _D), lambda e, r: (e, 0, r, 0)),
          pl.BlockSpec((1, 2 * _D, 2 * _D), lambda e, r: (e, 0, 0)),
          pl.BlockSpec((1, 2 * _D), lambda e, r: (e, 0)),
          pl.BlockSpec((1, 2 * _D, _D), lambda e, r: (e, 0, 0)),
          pl.BlockSpec((1, _D), lambda e, r: (e, 0)),
      ],
      out_specs=pl.BlockSpec((1, _RB, _D), lambda e, r: (e, r, 0)),
      out_shape=jax.ShapeDtypeStruct((2, _N, _D), jnp.float32),
  )(h1, agg2, w1, b1, w2, b2)


def _proj(h, w1, b1, g, be, w2, b2):
  return pl.pallas_call(
      _proj_body,
      grid=(2,),
      in_specs=[
          pl.BlockSpec((1, _N, _D), lambda e: (e, 0, 0)),
          pl.BlockSpec((1, _D, _D), lambda e: (e, 0, 0)),
          pl.BlockSpec((1, _D), lambda e: (e, 0)),
          pl.BlockSpec((1, _D), lambda e: (e, 0)),
          pl.BlockSpec((1, _D), lambda e: (e, 0)),
          pl.BlockSpec((1, _D, _D), lambda e: (e, 0, 0)),
          pl.BlockSpec((1, _D), lambda e: (e, 0)),
      ],
      out_specs=pl.BlockSpec((1, _N, _D), lambda e: (e, 0, 0)),
      out_shape=jax.ShapeDtypeStruct((2, _N, _D), jnp.float32),
  )(h, w1, b1, g, be, w2, b2)


def _stack(a, b, key):
  return jnp.stack([a[key], b[key]])


def kernel(x_phys, x_sem, edge_index, params):
  src = edge_index[0]
  dst = edge_index[1]
  pe = params['phys_enc']
  se = params['sem_enc']
  pp = params['phys_proj']
  sp = params['sem_proj']

  x_st = jnp.stack([x_phys, x_sem])

  agg1 = _segsum2(x_st.reshape(2 * _N, _D), src, dst).reshape(2, _N, _D)

  h1 = _l1(x_st, agg1,
           _stack(pe[0], se[0], 'W1'), _stack(pe[0], se[0], 'b1'),
           _stack(pe[0], se[0], 'W2'), _stack(pe[0], se[0], 'b2'))

  agg2_p = _segsum2(h1[0].reshape(2 * _N, _D), src, dst)
  agg2_s = _segsum2(h1[1].reshape(2 * _N, _D), src, dst)
  agg2 = jnp.stack([agg2_p.reshape(2, _N, _D), agg2_s.reshape(2, _N, _D)])

  h = _l2(h1, agg2,
          _stack(pe[1], se[1], 'W1'), _stack(pe[1], se[1], 'b1'),
          _stack(pe[1], se[1], 'W2'), _stack(pe[1], se[1], 'b2'))

  z = _proj(h,
            _stack(pp, sp, 'W1'), _stack(pp, sp, 'b1'),
            _stack(pp, sp, 'gamma'), _stack(pp, sp, 'beta'),
            _stack(pp, sp, 'W2'), _stack(pp, sp, 'b2'))

  return (h[0], h[1], z[0], z[1])


# speedup vs baseline: 2.4927x; 2.4927x over previous
import functools

import jax
import jax.numpy as jnp
from jax import lax
from jax.experimental import pallas as pl
from jax.experimental.pallas import tpu as pltpu
from jax.experimental.pallas import tpu_sc as plsc

_N = 10000
_E = 320000
_D = 128
_NPAD = 10240
_K = 80
_NSUB = 16
_EPS = _E // _NSUB
_NWIN = _EPS // _K
_ZROWS = _NPAD // _NSUB
_RB = 2000


def _segsum_body(tab_hbm, src_hbm, dst_hbm, out_hbm, srcv, dstv, rows, acc,
                 sem):
  c = lax.axis_index("c")
  s = lax.axis_index("s")

  zero = jnp.zeros((16,), jnp.float32)

  @pl.loop(0, _K)
  def _(r):
    @pl.loop(0, _D, step=16)
    def _(col):
      rows[r, pl.ds(col, 16)] = zero

  @pl.loop(0, _ZROWS, step=_K)
  def _(r0):
    pltpu.sync_copy(rows, acc.at[pl.ds(s * _ZROWS + r0, _K)])

  plsc.subcore_barrier()

  row_off = c * _N

  @pl.loop(0, _NWIN)
  def _(w):
    base = s * _EPS + w * _K
    pltpu.sync_copy(src_hbm.at[pl.ds(base, _K)], srcv.at[0])
    pltpu.sync_copy(dst_hbm.at[pl.ds(base, _K)], dstv.at[0])

    @pl.loop(0, _K, step=16)
    def _(j):
      srcv[0, pl.ds(j, 16)] = srcv[0, pl.ds(j, 16)] + row_off

    pltpu.async_copy(tab_hbm.at[srcv.at[0]], rows, sem).wait()
    pltpu.sync_copy(rows, acc.at[dstv.at[0]], add=True)

  plsc.subcore_barrier()
  pltpu.sync_copy(acc.at[pl.ds(s * _ZROWS, _ZROWS)],
                  out_hbm.at[pl.ds(c * _NPAD + s * _ZROWS, _ZROWS)])


def _segsum2(tables, src, dst):
  kern = pl.kernel(
      _segsum_body,
      out_type=jax.ShapeDtypeStruct((2 * _NPAD, _D), jnp.float32),
      mesh=plsc.VectorSubcoreMesh(core_axis_name="c", subcore_axis_name="s"),
      scratch_types=[
          pltpu.VMEM((1, _K), jnp.int32),
          pltpu.VMEM((1, _K), jnp.int32),
          pltpu.VMEM((_K, _D), jnp.float32),
          pltpu.VMEM_SHARED((_NPAD, _D), jnp.float32),
          pltpu.SemaphoreType.DMA,
      ],
  )
  return kern(tables, src, dst).reshape(2, _NPAD, _D)[:, :_N]


def _l1_body(x_ref, agg_ref, w1_ref, b1_ref, w2_ref, b2_ref, out_ref):
  m = x_ref[0] + agg_ref[0]
  u = jnp.maximum(
      jnp.dot(m, w1_ref[0], preferred_element_type=jnp.float32) + b1_ref[0],
      0.0)
  h = jnp.maximum(
      jnp.dot(u, w2_ref[0], preferred_element_type=jnp.float32) + b2_ref[0],
      0.0)
  out_ref[0, 0] = h[:, :_D]
  out_ref[0, 1] = h[:, _D:]


def _l2_body(h1_ref, agg_ref, w1_ref, b1_ref, w2_ref, b2_ref, out_ref):
  m_lo = h1_ref[0, 0] + agg_ref[0, 0]
  m_hi = h1_ref[0, 1] + agg_ref[0, 1]
  u = (jnp.dot(m_lo, w1_ref[0, :_D], preferred_element_type=jnp.float32)
       + jnp.dot(m_hi, w1_ref[0, _D:], preferred_element_type=jnp.float32)
       + b1_ref[0])
  u = jnp.maximum(u, 0.0)
  out_ref[0] = (jnp.dot(u, w2_ref[0], preferred_element_type=jnp.float32)
                + b2_ref[0])


def _proj_body(h_ref, w1_ref, b1_ref, g_ref, be_ref, w2_ref, b2_ref, out_ref):
  u = jnp.dot(h_ref[0], w1_ref[0], preferred_element_type=jnp.float32) \
      + b1_ref[0]
  mean = jnp.mean(u, axis=0)
  var = jnp.mean((u - mean) ** 2, axis=0)
  un = g_ref[0] * (u - mean) / jnp.sqrt(var + 1e-5) + be_ref[0]
  ur = jnp.maximum(un, 0.0)
  out_ref[0] = (jnp.dot(ur, w2_ref[0], preferred_element_type=jnp.float32)
                + b2_ref[0])


def _l1(x_st, agg1, w1, b1, w2, b2):
  grid = (2, _N // _RB)
  return pl.pallas_call(
      _l1_body,
      grid=grid,
      in_specs=[
          pl.BlockSpec((1, _RB, _D), lambda e, r: (e, r, 0)),
          pl.BlockSpec((1, _RB, _D), lambda e, r: (e, r, 0)),
          pl.BlockSpec((1, _D, 2 * _D), lambda e, r: (e, 0, 0)),
          pl.BlockSpec((1, 1, 2 * _D), lambda e, r: (e, 0, 0)),
          pl.BlockSpec((1, 2 * _D, 2 * _D), lambda e, r: (e, 0, 0)),
          pl.BlockSpec((1, 1, 2 * _D), lambda e, r: (e, 0, 0)),
      ],
      out_specs=pl.BlockSpec((1, 2, _RB, _D), lambda e, r: (e, 0, r, 0)),
      out_shape=jax.ShapeDtypeStruct((2, 2, _N, _D), jnp.float32),
  )(x_st, agg1, w1, b1, w2, b2)


def _l2(h1, agg2, w1, b1, w2, b2):
  grid = (2, _N // _RB)
  return pl.pallas_call(
      _l2_body,
      grid=grid,
      in_specs=[
          pl.BlockSpec((1, 2, _RB, _D), lambda e, r: (e, 0, r, 0)),
          pl.BlockSpec((1, 2, _RB, _D), lambda e, r: (e, 0, r, 0)),
          pl.BlockSpec((1, 2 * _D, 2 * _D), lambda e, r: (e, 0, 0)),
          pl.BlockSpec((1, 1, 2 * _D), lambda e, r: (e, 0, 0)),
          pl.BlockSpec((1, 2 * _D, _D), lambda e, r: (e, 0, 0)),
          pl.BlockSpec((1, 1, _D), lambda e, r: (e, 0, 0)),
      ],
      out_specs=pl.BlockSpec((1, _RB, _D), lambda e, r: (e, r, 0)),
      out_shape=jax.ShapeDtypeStruct((2, _N, _D), jnp.float32),
  )(h1, agg2, w1, b1, w2, b2)


def _proj(h, w1, b1, g, be, w2, b2):
  return pl.pallas_call(
      _proj_body,
      grid=(2,),
      in_specs=[
          pl.BlockSpec((1, _N, _D), lambda e: (e, 0, 0)),
          pl.BlockSpec((1, _D, _D), lambda e: (e, 0, 0)),
          pl.BlockSpec((1, 1, _D), lambda e: (e, 0, 0)),
          pl.BlockSpec((1, 1, _D), lambda e: (e, 0, 0)),
          pl.BlockSpec((1, 1, _D), lambda e: (e, 0, 0)),
          pl.BlockSpec((1, _D, _D), lambda e: (e, 0, 0)),
          pl.BlockSpec((1, 1, _D), lambda e: (e, 0, 0)),
      ],
      out_specs=pl.BlockSpec((1, _N, _D), lambda e: (e, 0, 0)),
      out_shape=jax.ShapeDtypeStruct((2, _N, _D), jnp.float32),
  )(h, w1, b1, g, be, w2, b2)


def _stack(a, b, key):
  return jnp.stack([a[key], b[key]])


def _stackv(a, b, key):
  return jnp.stack([a[key], b[key]])[:, None, :]


def kernel(x_phys, x_sem, edge_index, params):
  src = edge_index[0]
  dst = edge_index[1]
  pe = params['phys_enc']
  se = params['sem_enc']
  pp = params['phys_proj']
  sp = params['sem_proj']

  x_st = jnp.stack([x_phys, x_sem])

  agg1 = _segsum2(x_st.reshape(2 * _N, _D), src, dst)

  h1 = _l1(x_st, agg1,
           _stack(pe[0], se[0], 'W1'), _stackv(pe[0], se[0], 'b1'),
           _stack(pe[0], se[0], 'W2'), _stackv(pe[0], se[0], 'b2'))

  agg2_p = _segsum2(h1[0].reshape(2 * _N, _D), src, dst)
  agg2_s = _segsum2(h1[1].reshape(2 * _N, _D), src, dst)
  agg2 = jnp.stack([agg2_p, agg2_s])

  h = _l2(h1, agg2,
          _stack(pe[1], se[1], 'W1'), _stackv(pe[1], se[1], 'b1'),
          _stack(pe[1], se[1], 'W2'), _stackv(pe[1], se[1], 'b2'))

  z = _proj(h,
            _stack(pp, sp, 'W1'), _stackv(pp, sp, 'b1'),
            _stackv(pp, sp, 'gamma'), _stackv(pp, sp, 'beta'),
            _stack(pp, sp, 'W2'), _stackv(pp, sp, 'b2'))

  return (h[0], h[1], z[0], z[1])


# speedup vs baseline: 6.1071x; 2.4500x over previous
import functools

import jax
import jax.numpy as jnp
from jax import lax
from jax.experimental import pallas as pl
from jax.experimental.pallas import tpu as pltpu
from jax.experimental.pallas import tpu_sc as plsc
_N = 10000
_E = 320000
_D = 128
_NPAD = 10240
_K = 64
_NSUB = 16
_EROWS = 5120
_EPAD = _EROWS * _K - _E
_NWIN = _EROWS // _NSUB
_NQ = _NWIN // 4
_ZROWS = _NPAD // _NSUB
_RB = 2000


def _segsum_body(t0_hbm, t1_hbm, e_hbm, out_hbm, ia, ib, r0, r1, r2, r3, acc,
                 sia, sib, sr0, sr1, sr2, sr3, ss0, ss1, ss2, ss3):
  c = lax.axis_index("c")
  s = lax.axis_index("s")
  bufs = (r0, r1, r2, r3)
  gsem = (sr0, sr1, sr2, sr3)
  ssem = (ss0, ss1, ss2, ss3)
  ebase = 2 * _NWIN * s

  def fire_idx(q, slot, sem):
    pltpu.async_copy(e_hbm.at[pl.ds(ebase + 8 * q, 8)], slot, sem)

  def wait_idx(slot, sem):
    pltpu.make_async_copy(e_hbm.at[pl.ds(0, 8)], slot, sem).wait()

  fire_idx(0, ia, sia)
  fire_idx(1, ib, sib)

  zero = jnp.zeros((16,), jnp.float32)

  @pl.loop(0, _K)
  def _(r):
    @pl.loop(0, _D, step=16)
    def _(col):
      r0[r, pl.ds(col, 16)] = zero

  @pl.loop(0, _ZROWS, step=_K)
  def _(z0):
    pltpu.sync_copy(r0, acc.at[pl.ds(s * _ZROWS + z0, _K)])

  def run(tab):
    def fire_gather(slot, j, buf, sem):
      pltpu.async_copy(tab.at[slot.at[2 * j]], buf, sem)

    def drain(buf, sem):
      pltpu.make_async_copy(tab.at[pl.ds(0, _K)], buf, sem).wait()

    def fire_scatter(slot, j, buf, sem):
      pltpu.async_copy(buf, acc.at[slot.at[2 * j + 1]], sem, add=True)

    def half(q, icur, inext, sem_icur, sem_inext, prefetch):
      for j in range(4):
        drain(bufs[j], gsem[j])
        fire_scatter(icur, j, bufs[j], ssem[j])
      wait_idx(inext, sem_inext)
      for j in range(4):
        drain(bufs[j], ssem[j])
        fire_gather(inext, j, bufs[j], gsem[j])
      if prefetch:
        fire_idx(q + 2, icur, sem_icur)

    wait_idx(ia, sia)
    for j in range(4):
      fire_gather(ia, j, bufs[j], gsem[j])
    plsc.subcore_barrier()

    @pl.loop(0, _NQ - 2, step=2)
    def _(q):
      half(q, ia, ib, sia, sib, True)
      half(q + 1, ib, ia, sib, sia, True)

    half(_NQ - 2, ia, ib, sia, sib, False)
    for j in range(4):
      drain(bufs[j], gsem[j])
      pltpu.sync_copy(bufs[j], acc.at[ib.at[2 * j + 1]], add=True)

  @pl.when(c == 0)
  def _():
    run(t0_hbm)

  @pl.when(c == 1)
  def _():
    run(t1_hbm)

  plsc.subcore_barrier()
  pltpu.sync_copy(acc.at[pl.ds(s * _ZROWS, _ZROWS)],
                  out_hbm.at[pl.ds(c * _NPAD + s * _ZROWS, _ZROWS)])


def _segsum2(tab0, tab1, edges):
  kern = pl.kernel(
      _segsum_body,
      out_type=jax.ShapeDtypeStruct((2 * _NPAD, _D), jnp.float32),
      mesh=plsc.VectorSubcoreMesh(core_axis_name="c", subcore_axis_name="s"),
      scratch_types=[
          pltpu.VMEM((8, _K), jnp.int32),
          pltpu.VMEM((8, _K), jnp.int32),
          pltpu.VMEM((_K, _D), jnp.float32),
          pltpu.VMEM((_K, _D), jnp.float32),
          pltpu.VMEM((_K, _D), jnp.float32),
          pltpu.VMEM((_K, _D), jnp.float32),
          pltpu.VMEM_SHARED((_NPAD, _D), jnp.float32),
      ] + [pltpu.SemaphoreType.DMA] * 10,
  )
  return kern(tab0, tab1, edges).reshape(2, _NPAD, _D)[:, :_N]


def _l1_body(x_ref, agg_ref, w1_ref, b1_ref, w2_ref, b2_ref, out_ref):
  m = x_ref[0] + agg_ref[0]
  u = jnp.maximum(
      jnp.dot(m, w1_ref[0], preferred_element_type=jnp.float32) + b1_ref[0],
      0.0)
  h = jnp.maximum(
      jnp.dot(u, w2_ref[0], preferred_element_type=jnp.float32) + b2_ref[0],
      0.0)
  out_ref[0, 0] = h[:, :_D]
  out_ref[0, 1] = h[:, _D:]


def _l2_body(h1_ref, agg_ref, w1_ref, b1_ref, w2_ref, b2_ref, out_ref):
  m_lo = h1_ref[0, 0] + agg_ref[0, 0]
  m_hi = h1_ref[0, 1] + agg_ref[0, 1]
  u = (jnp.dot(m_lo, w1_ref[0, :_D], preferred_element_type=jnp.float32)
       + jnp.dot(m_hi, w1_ref[0, _D:], preferred_element_type=jnp.float32)
       + b1_ref[0])
  u = jnp.maximum(u, 0.0)
  out_ref[0] = (jnp.dot(u, w2_ref[0], preferred_element_type=jnp.float32)
                + b2_ref[0])


def _proj_body(h_ref, w1_ref, b1_ref, g_ref, be_ref, w2_ref, b2_ref, out_ref):
  u = jnp.dot(h_ref[0], w1_ref[0], preferred_element_type=jnp.float32) \
      + b1_ref[0]
  mean = jnp.mean(u, axis=0)
  var = jnp.mean((u - mean) ** 2, axis=0)
  un = g_ref[0] * (u - mean) / jnp.sqrt(var + 1e-5) + be_ref[0]
  ur = jnp.maximum(un, 0.0)
  out_ref[0] = (jnp.dot(ur, w2_ref[0], preferred_element_type=jnp.float32)
                + b2_ref[0])


def _l1(x_st, agg1, w1, b1, w2, b2):
  grid = (2, _N // _RB)
  return pl.pallas_call(
      _l1_body,
      grid=grid,
      in_specs=[
          pl.BlockSpec((1, _RB, _D), lambda e, r: (e, r, 0)),
          pl.BlockSpec((1, _RB, _D), lambda e, r: (e, r, 0)),
          pl.BlockSpec((1, _D, 2 * _D), lambda e, r: (e, 0, 0)),
          pl.BlockSpec((1, 1, 2 * _D), lambda e, r: (e, 0, 0)),
          pl.BlockSpec((1, 2 * _D, 2 * _D), lambda e, r: (e, 0, 0)),
          pl.BlockSpec((1, 1, 2 * _D), lambda e, r: (e, 0, 0)),
      ],
      out_specs=pl.BlockSpec((1, 2, _RB, _D), lambda e, r: (e, 0, r, 0)),
      out_shape=jax.ShapeDtypeStruct((2, 2, _N, _D), jnp.float32),
  )(x_st, agg1, w1, b1, w2, b2)


def _l2(h1, agg2, w1, b1, w2, b2):
  grid = (2, _N // _RB)
  return pl.pallas_call(
      _l2_body,
      grid=grid,
      in_specs=[
          pl.BlockSpec((1, 2, _RB, _D), lambda e, r: (e, 0, r, 0)),
          pl.BlockSpec((1, 2, _RB, _D), lambda e, r: (e, 0, r, 0)),
          pl.BlockSpec((1, 2 * _D, 2 * _D), lambda e, r: (e, 0, 0)),
          pl.BlockSpec((1, 1, 2 * _D), lambda e, r: (e, 0, 0)),
          pl.BlockSpec((1, 2 * _D, _D), lambda e, r: (e, 0, 0)),
          pl.BlockSpec((1, 1, _D), lambda e, r: (e, 0, 0)),
      ],
      out_specs=pl.BlockSpec((1, _RB, _D), lambda e, r: (e, r, 0)),
      out_shape=jax.ShapeDtypeStruct((2, _N, _D), jnp.float32),
  )(h1, agg2, w1, b1, w2, b2)


def _proj(h, w1, b1, g, be, w2, b2):
  return pl.pallas_call(
      _proj_body,
      grid=(2,),
      in_specs=[
          pl.BlockSpec((1, _N, _D), lambda e: (e, 0, 0)),
          pl.BlockSpec((1, _D, _D), lambda e: (e, 0, 0)),
          pl.BlockSpec((1, 1, _D), lambda e: (e, 0, 0)),
          pl.BlockSpec((1, 1, _D), lambda e: (e, 0, 0)),
          pl.BlockSpec((1, 1, _D), lambda e: (e, 0, 0)),
          pl.BlockSpec((1, _D, _D), lambda e: (e, 0, 0)),
          pl.BlockSpec((1, 1, _D), lambda e: (e, 0, 0)),
      ],
      out_specs=pl.BlockSpec((1, _N, _D), lambda e: (e, 0, 0)),
      out_shape=jax.ShapeDtypeStruct((2, _N, _D), jnp.float32),
  )(h, w1, b1, g, be, w2, b2)


def _stack(a, b, key):
  return jnp.stack([a[key], b[key]])


def _stackv(a, b, key):
  return jnp.stack([a[key], b[key]])[:, None, :]


def kernel(x_phys, x_sem, edge_index, params):
  it = jnp.arange(_EPAD, dtype=jnp.int32)
  src = jnp.concatenate([edge_index[0], (it * 13) % _N]).reshape(_EROWS, _K)
  dst = jnp.concatenate([edge_index[1], _N + (it % (_NPAD - _N))]).reshape(
      _EROWS, _K)
  edges = jnp.stack([src, dst], axis=1).reshape(2 * _EROWS, _K)
  pe = params['phys_enc']
  se = params['sem_enc']
  pp = params['phys_proj']
  sp = params['sem_proj']

  x_st = jnp.stack([x_phys, x_sem])

  agg1 = _segsum2(x_phys, x_sem, edges)

  h1 = _l1(x_st, agg1,
           _stack(pe[0], se[0], 'W1'), _stackv(pe[0], se[0], 'b1'),
           _stack(pe[0], se[0], 'W2'), _stackv(pe[0], se[0], 'b2'))

  agg2_p = _segsum2(h1[0, 0], h1[0, 1], edges)
  agg2_s = _segsum2(h1[1, 0], h1[1, 1], edges)
  agg2 = jnp.stack([agg2_p, agg2_s])

  h = _l2(h1, agg2,
          _stack(pe[1], se[1], 'W1'), _stackv(pe[1], se[1], 'b1'),
          _stack(pe[1], se[1], 'W2'), _stackv(pe[1], se[1], 'b2'))

  z = _proj(h,
            _stack(pp, sp, 'W1'), _stackv(pp, sp, 'b1'),
            _stackv(pp, sp, 'gamma'), _stackv(pp, sp, 'beta'),
            _stack(pp, sp, 'W2'), _stackv(pp, sp, 'b2'))

  return (h[0], h[1], z[0], z[1])
